# Initial kernel scaffold; baseline (speedup 1.0000x reference)
import jax
import jax.numpy as jnp
from jax.experimental import pallas as pl


def kernel(x, W):
    raise NotImplementedError("write your pallas kernel here")


# speedup vs baseline: 2.5504x; 2.5504x over previous
import functools

import jax
import jax.numpy as jnp
from jax import lax
from jax.experimental import pallas as pl
from jax.experimental.pallas import tpu as pltpu
from jax.experimental.pallas import tpu_sc as plsc

_VOCAB = 100000
_HIDDEN = 128
_BATCH = 4096
_TEXT_LEN = 200

_NUM_WORKERS = 32
_IDX_ROWS = 4
_CHUNK = _IDX_ROWS * 128


def _sc_gather(x_flat, W):
  n = x_flat.shape[0]
  per_w = n // _NUM_WORKERS
  steps = per_w // _CHUNK
  assert per_w % _CHUNK == 0

  mesh = plsc.VectorSubcoreMesh(core_axis_name="c", subcore_axis_name="s")

  @functools.partial(
      pl.kernel,
      out_type=jax.ShapeDtypeStruct((n, _HIDDEN), jnp.float32),
      mesh=mesh,
      scratch_types=[
          pltpu.VMEM((_CHUNK,), jnp.int32),
          pltpu.VMEM((_CHUNK, _HIDDEN), jnp.float32),
          pltpu.SemaphoreType.DMA,
      ],
  )
  def k(w_hbm, x_hbm, out_hbm, idx_v, rows_v, sem):
    wid = lax.axis_index("s") * 2 + lax.axis_index("c")
    base = wid * per_w

    def step(i, carry):
      off = base + i * _CHUNK
      pltpu.sync_copy(x_hbm.at[pl.ds(off, _CHUNK)], idx_v)
      copies = []
      for j in range(_IDX_ROWS):
        copies.append(
            pltpu.async_copy(
                w_hbm.at[idx_v.at[pl.ds(j * 128, 128)]],
                rows_v.at[pl.ds(j * 128, 128)],
                sem,
            ))
      for c in copies:
        c.wait()
      pltpu.sync_copy(rows_v, out_hbm.at[pl.ds(off, _CHUNK)])
      return carry

    lax.fori_loop(0, steps, step, 0)

  return k(W, x_flat)


def _tc_transpose(g):
  B, L, H = g.shape
  BB = 16

  def body(g_ref, o_ref):
    o_ref[...] = jnp.transpose(g_ref[...], (0, 2, 1))

  return pl.pallas_call(
      body,
      grid=(B // BB,),
      in_specs=[pl.BlockSpec((BB, L, H), lambda i: (i, 0, 0))],
      out_specs=pl.BlockSpec((BB, H, L), lambda i: (i, 0, 0)),
      out_shape=jax.ShapeDtypeStruct((B, H, L), jnp.float32),
  )(g)


@jax.jit
def kernel(x, W):
  xf = x.reshape(-1).astype(jnp.int32)
  g = _sc_gather(xf, W)
  g3 = g.reshape(_BATCH, _TEXT_LEN, _HIDDEN)
  return _tc_transpose(g3)


# speedup vs baseline: 2.7386x; 1.0738x over previous
import functools

import jax
import jax.numpy as jnp
from jax import lax
from jax.experimental import pallas as pl
from jax.experimental.pallas import tpu as pltpu
from jax.experimental.pallas import tpu_sc as plsc

_VOCAB = 100000
_HIDDEN = 128
_BATCH = 4096
_TEXT_LEN = 200

_NUM_WORKERS = 32
_IDX_ROWS = 4
_CHUNK = _IDX_ROWS * 128


def _sc_gather(x_flat, W):
  n = x_flat.shape[0]
  per_w = n // _NUM_WORKERS
  steps = per_w // _CHUNK
  assert per_w % _CHUNK == 0

  mesh = plsc.VectorSubcoreMesh(core_axis_name="c", subcore_axis_name="s")

  @functools.partial(
      pl.kernel,
      out_type=jax.ShapeDtypeStruct((n, _HIDDEN), jnp.float32),
      mesh=mesh,
      scratch_types=[
          pltpu.VMEM((_CHUNK,), jnp.int32),
          pltpu.VMEM((_CHUNK, _HIDDEN), jnp.float32),
          pltpu.SemaphoreType.DMA,
      ],
  )
  def k(w_hbm, x_hbm, out_hbm, idx_v, rows_v, sem):
    wid = lax.axis_index("s") * 2 + lax.axis_index("c")
    base = wid * per_w

    def step(i, carry):
      off = base + i * _CHUNK
      pltpu.sync_copy(x_hbm.at[pl.ds(off, _CHUNK)], idx_v)
      copies = []
      for j in range(_IDX_ROWS):
        copies.append(
            pltpu.async_copy(
                w_hbm.at[idx_v.at[pl.ds(j * 128, 128)]],
                rows_v.at[pl.ds(j * 128, 128)],
                sem,
            ))
      for c in copies:
        c.wait()
      pltpu.sync_copy(rows_v, out_hbm.at[pl.ds(off, _CHUNK)])
      return carry

    lax.fori_loop(0, steps, step, 0)

  return k(W, x_flat)


def _tc_transpose(g):
  B, L, H = g.shape
  BB = 64

  def body(g_ref, o_ref):
    o_ref[...] = jnp.transpose(g_ref[...], (0, 2, 1))

  return pl.pallas_call(
      body,
      grid=(B // BB,),
      in_specs=[pl.BlockSpec((BB, L, H), lambda i: (i, 0, 0))],
      out_specs=pl.BlockSpec((BB, H, L), lambda i: (i, 0, 0)),
      out_shape=jax.ShapeDtypeStruct((B, H, L), jnp.float32),
  )(g)


@jax.jit
def kernel(x, W):
  xf = x.reshape(-1).astype(jnp.int32)
  g = _sc_gather(xf, W)
  g3 = g.reshape(_BATCH, _TEXT_LEN, _HIDDEN)
  return _tc_transpose(g3)


# speedup vs baseline: 2.7453x; 1.0025x over previous
import functools

import jax
import jax.numpy as jnp
from jax import lax
from jax.experimental import pallas as pl
from jax.experimental.pallas import tpu as pltpu
from jax.experimental.pallas import tpu_sc as plsc

_VOCAB = 100000
_HIDDEN = 128
_BATCH = 4096
_TEXT_LEN = 200

_NUM_WORKERS = 32
_IDX_ROWS = 4
_CHUNK = _IDX_ROWS * 128


def _sc_gather(x_flat, W):
  n = x_flat.shape[0]
  per_w = n // _NUM_WORKERS
  steps = per_w // _CHUNK
  assert per_w % _CHUNK == 0

  mesh = plsc.VectorSubcoreMesh(core_axis_name="c", subcore_axis_name="s")

  @functools.partial(
      pl.kernel,
      out_type=jax.ShapeDtypeStruct((n, _HIDDEN), jnp.float32),
      mesh=mesh,
      scratch_types=[
          pltpu.VMEM((_CHUNK,), jnp.int32),
          pltpu.VMEM((_CHUNK, _HIDDEN), jnp.float32),
          pltpu.SemaphoreType.DMA,
      ],
  )
  def k(w_hbm, x_hbm, out_hbm, idx_v, rows_v, sem):
    wid = lax.axis_index("s") * 2 + lax.axis_index("c")
    base = wid * per_w

    def step(i, carry):
      off = base + i * _CHUNK
      pltpu.sync_copy(x_hbm.at[pl.ds(off, _CHUNK)], idx_v)
      copies = []
      for j in range(_IDX_ROWS):
        copies.append(
            pltpu.async_copy(
                w_hbm.at[idx_v.at[pl.ds(j * 128, 128)]],
                rows_v.at[pl.ds(j * 128, 128)],
                sem,
            ))
      for c in copies:
        c.wait()
      pltpu.sync_copy(rows_v, out_hbm.at[pl.ds(off, _CHUNK)])
      return carry

    lax.fori_loop(0, steps, step, 0)

  return k(W, x_flat)


def _tc_transpose(g):
  B, L, H = g.shape
  BB = 128

  def body(g_ref, o_ref):
    o_ref[...] = jnp.transpose(g_ref[...], (0, 2, 1))

  return pl.pallas_call(
      body,
      grid=(B // BB,),
      in_specs=[pl.BlockSpec((BB, L, H), lambda i: (i, 0, 0))],
      out_specs=pl.BlockSpec((BB, H, L), lambda i: (i, 0, 0)),
      out_shape=jax.ShapeDtypeStruct((B, H, L), jnp.float32),
  )(g)


@jax.jit
def kernel(x, W):
  xf = x.reshape(-1).astype(jnp.int32)
  g = _sc_gather(xf, W)
  g3 = g.reshape(_BATCH, _TEXT_LEN, _HIDDEN)
  return _tc_transpose(g3)


# speedup vs baseline: 2.8219x; 1.0279x over previous
import functools

import jax
import jax.numpy as jnp
from jax import lax
from jax.experimental import pallas as pl
from jax.experimental.pallas import tpu as pltpu
from jax.experimental.pallas import tpu_sc as plsc

_VOCAB = 100000
_HIDDEN = 128
_BATCH = 4096
_TEXT_LEN = 200

_NUM_WORKERS = 32
_IDX_ROWS = 4
_CHUNK = _IDX_ROWS * 128

_N_HALF = 2
_HB = _BATCH // _N_HALF
_BB = 128
_NB = _HB // _BB


def _sc_gather(x_flat, W):
  n = x_flat.shape[0]
  per_w = n // _NUM_WORKERS
  steps = per_w // _CHUNK
  assert per_w % _CHUNK == 0

  mesh = plsc.VectorSubcoreMesh(core_axis_name="c", subcore_axis_name="s")

  @functools.partial(
      pl.kernel,
      out_type=jax.ShapeDtypeStruct((n, _HIDDEN), jnp.float32),
      mesh=mesh,
      scratch_types=[
          pltpu.VMEM((_CHUNK,), jnp.int32),
          pltpu.VMEM((_CHUNK, _HIDDEN), jnp.float32),
          pltpu.SemaphoreType.DMA,
      ],
  )
  def k(w_hbm, x_hbm, out_hbm, idx_v, rows_v, sem):
    wid = lax.axis_index("s") * 2 + lax.axis_index("c")
    base = wid * per_w

    def step(i, carry):
      off = base + i * _CHUNK
      pltpu.sync_copy(x_hbm.at[pl.ds(off, _CHUNK)], idx_v)
      copies = []
      for j in range(_IDX_ROWS):
        copies.append(
            pltpu.async_copy(
                w_hbm.at[idx_v.at[pl.ds(j * 128, 128)]],
                rows_v.at[pl.ds(j * 128, 128)],
                sem,
            ))
      for c in copies:
        c.wait()
      pltpu.sync_copy(rows_v, out_hbm.at[pl.ds(off, _CHUNK)])
      return carry

    lax.fori_loop(0, steps, step, 0)

  return k(W, x_flat)


def _transpose_first(g):
  def body(g_ref, o_ref):
    o_ref[...] = jnp.transpose(g_ref[...], (0, 2, 1))

  return pl.pallas_call(
      body,
      grid=(_NB,),
      in_specs=[pl.BlockSpec((_BB, _TEXT_LEN, _HIDDEN), lambda i: (i, 0, 0))],
      out_specs=pl.BlockSpec((_BB, _HIDDEN, _TEXT_LEN), lambda i: (i, 0, 0)),
      out_shape=jax.ShapeDtypeStruct((_BATCH, _HIDDEN, _TEXT_LEN),
                                     jnp.float32),
  )(g)


def _transpose_second(buf, g):
  def body(buf_ref, g_ref, o_ref):
    del buf_ref
    o_ref[...] = jnp.transpose(g_ref[...], (0, 2, 1))

  return pl.pallas_call(
      body,
      grid=(_NB,),
      in_specs=[
          pl.BlockSpec(memory_space=pl.ANY),
          pl.BlockSpec((_BB, _TEXT_LEN, _HIDDEN), lambda i: (i, 0, 0)),
      ],
      out_specs=pl.BlockSpec((_BB, _HIDDEN, _TEXT_LEN),
                             lambda i: (i + _NB, 0, 0)),
      out_shape=jax.ShapeDtypeStruct((_BATCH, _HIDDEN, _TEXT_LEN),
                                     jnp.float32),
      input_output_aliases={0: 0},
  )(buf, g)


@jax.jit
def kernel(x, W):
  xi = x.astype(jnp.int32)
  x0 = xi[:_HB].reshape(-1)
  x1 = xi[_HB:].reshape(-1)
  g0 = _sc_gather(x0, W).reshape(_HB, _TEXT_LEN, _HIDDEN)
  g1 = _sc_gather(x1, W).reshape(_HB, _TEXT_LEN, _HIDDEN)
  buf = _transpose_first(g0)
  return _transpose_second(buf, g1)
